# Initial kernel scaffold; baseline (speedup 1.0000x reference)
import jax
import jax.numpy as jnp
from jax.experimental import pallas as pl


def kernel(token_ids, table):
    raise NotImplementedError("write your pallas kernel here")


# speedup vs baseline: 1.6253x; 1.6253x over previous
import functools

import jax
import jax.numpy as jnp
from jax import lax
from jax.experimental import pallas as pl
from jax.experimental.pallas import tpu as pltpu
from jax.experimental.pallas import tpu_sc as plsc

_NUM_CORES = 2
_NUM_SUBCORES = 16
_NUM_WORKERS = _NUM_CORES * _NUM_SUBCORES
_CHUNK = 32


def _emb_kernel(n_chunks, chunk, n_per_w, d_model, ids_hbm, table_hbm,
                out_hbm, idx_v, rows_a, rows_b, sem_a, sem_b):
    cid = lax.axis_index("c")
    sid = lax.axis_index("s")
    wid = sid * _NUM_CORES + cid
    base = wid * n_per_w

    pltpu.sync_copy(ids_hbm.at[wid], idx_v)

    bufs = (rows_a, rows_b)
    sems = (sem_a, sem_b)

    copy0 = pltpu.async_copy(table_hbm.at[idx_v.at[0]], bufs[0], sems[0])
    prev = copy0
    for c in range(n_chunks):
        nxt = None
        if c + 1 < n_chunks:
            nb = (c + 1) % 2
            nxt = pltpu.async_copy(
                table_hbm.at[idx_v.at[c + 1]], bufs[nb], sems[nb])
        prev.wait()
        pltpu.sync_copy(bufs[c % 2], out_hbm.at[pl.ds(base + c * chunk, chunk)])
        prev = nxt


def kernel(token_ids, table):
    b, t = token_ids.shape
    v, d = table.shape
    n = b * t
    n_per_w = n // _NUM_WORKERS
    chunk = _CHUNK
    n_chunks = n_per_w // chunk

    ids = token_ids.reshape(_NUM_WORKERS, n_chunks, chunk).astype(jnp.int32)

    mesh = plsc.VectorSubcoreMesh(core_axis_name="c", subcore_axis_name="s")
    emb = functools.partial(
        pl.kernel,
        mesh=mesh,
        out_type=jax.ShapeDtypeStruct((n, d), jnp.float32),
        scratch_types=[
            pltpu.VMEM((n_chunks, chunk), jnp.int32),
            pltpu.VMEM((chunk, d), jnp.float32),
            pltpu.VMEM((chunk, d), jnp.float32),
            pltpu.SemaphoreType.DMA,
            pltpu.SemaphoreType.DMA,
        ],
    )(functools.partial(_emb_kernel, n_chunks, chunk, n_per_w, d))

    out = emb(ids, table)
    return out.reshape(b, t, d)


# speedup vs baseline: 1.6518x; 1.0163x over previous
import functools

import jax
import jax.numpy as jnp
from jax import lax
from jax.experimental import pallas as pl
from jax.experimental.pallas import tpu as pltpu
from jax.experimental.pallas import tpu_sc as plsc

_NUM_CORES = 2
_NUM_SUBCORES = 16
_NUM_WORKERS = _NUM_CORES * _NUM_SUBCORES
_CHUNK = 32
_NBUF = 3


def _emb_kernel(n_chunks, chunk, n_per_w, ids_hbm, table_hbm, out_hbm,
                idx_v, rows_a, rows_b, rows_c,
                gsem_a, gsem_b, gsem_c, ssem_a, ssem_b, ssem_c):
    cid = lax.axis_index("c")
    sid = lax.axis_index("s")
    wid = sid * _NUM_CORES + cid
    base = wid * n_per_w

    pltpu.sync_copy(ids_hbm.at[wid], idx_v)

    bufs = (rows_a, rows_b, rows_c)
    gsems = (gsem_a, gsem_b, gsem_c)
    ssems = (ssem_a, ssem_b, ssem_c)

    def gather(c):
        b = c % _NBUF
        return pltpu.async_copy(table_hbm.at[idx_v.at[c]], bufs[b], gsems[b])

    def store(c):
        b = c % _NBUF
        return pltpu.async_copy(
            bufs[b], out_hbm.at[pl.ds(base + c * chunk, chunk)], ssems[b])

    gathers = [None] * n_chunks
    stores = [None] * n_chunks
    for b in range(min(_NBUF, n_chunks)):
        gathers[b] = gather(b)
    for c in range(n_chunks):
        if c >= 1 and c - 1 + _NBUF < n_chunks:
            stores[c - 1].wait()
            gathers[c - 1 + _NBUF] = gather(c - 1 + _NBUF)
        gathers[c].wait()
        stores[c] = store(c)
    for c in range(max(0, n_chunks - _NBUF), n_chunks):
        stores[c].wait()


def kernel(token_ids, table):
    b, t = token_ids.shape
    v, d = table.shape
    n = b * t
    n_per_w = n // _NUM_WORKERS
    chunk = _CHUNK
    n_chunks = n_per_w // chunk

    ids = token_ids.reshape(_NUM_WORKERS, n_chunks, chunk).astype(jnp.int32)

    mesh = plsc.VectorSubcoreMesh(core_axis_name="c", subcore_axis_name="s")
    emb = functools.partial(
        pl.kernel,
        mesh=mesh,
        out_type=jax.ShapeDtypeStruct((n, d), jnp.float32),
        scratch_types=[
            pltpu.VMEM((n_chunks, chunk), jnp.int32),
            pltpu.VMEM((chunk, d), jnp.float32),
            pltpu.VMEM((chunk, d), jnp.float32),
            pltpu.VMEM((chunk, d), jnp.float32),
            pltpu.SemaphoreType.DMA,
            pltpu.SemaphoreType.DMA,
            pltpu.SemaphoreType.DMA,
            pltpu.SemaphoreType.DMA,
            pltpu.SemaphoreType.DMA,
            pltpu.SemaphoreType.DMA,
        ],
    )(functools.partial(_emb_kernel, n_chunks, chunk, n_per_w))

    out = emb(ids, table)
    return out.reshape(b, t, d)
